# baseline (speedup 1.0000x reference)
import jax
import jax.numpy as jnp
from jax.experimental import pallas as pl

N = 50000
EMB = 16
HID = 32
HEADS = 4
G = 64


def _emb_feat(table, idx):
    e = jnp.take(table, idx, axis=0)
    m = (idx != 0)[..., None].astype(jnp.float32)
    return (e * m).sum(axis=1) / (m.sum(axis=1) + 1e-9)


def _layernorm(x, g, b):
    mu = x.mean(-1, keepdims=True)
    var = ((x - mu) ** 2).mean(-1, keepdims=True)
    return (x - mu) / jnp.sqrt(var + 1e-5) * g + b


def _gat(x, src, dst, W, a_s, a_d, bias, heads, out_ch):
    n = x.shape[0]
    loop = jnp.arange(n, dtype=src.dtype)
    s = jnp.concatenate([src, loop])
    d = jnp.concatenate([dst, loop])
    xw = (x @ W).reshape(n, heads, out_ch)
    al_s = (xw * a_s[None]).sum(-1)
    al_d = (xw * a_d[None]).sum(-1)
    e = jax.nn.leaky_relu(al_s[s] + al_d[d], 0.2)
    emax = jax.ops.segment_max(e, d, num_segments=n)
    ex = jnp.exp(e - emax[d])
    den = jax.ops.segment_sum(ex, d, num_segments=n)
    alpha = ex / (den[d] + 1e-16)
    out = jax.ops.segment_sum(xw[s] * alpha[..., None], d, num_segments=n)
    out = out.reshape(n, heads * out_ch)
    return out + bias


def _mlp_kernel(pooled_ref, w1_ref, b1_ref, w2_ref, b2_ref, out_ref):
    h = jnp.maximum(pooled_ref[...] @ w1_ref[...] + b1_ref[...], 0.0)
    out_ref[...] = h @ w2_ref[...] + b2_ref[...]


def kernel(x_scalar, x_opcode, x_source, x_sink, x_string_manip, x_payload, edge_index, batch, emb_opcode, emb_source, emb_sink, emb_strman, emb_payload, ln_g, ln_b, W1, att_src1, att_dst1, b1, W2, att_src2, att_dst2, b2, fc1_w, fc1_b, fc2_w, fc2_b):
    feats = jnp.concatenate([
        x_scalar,
        _emb_feat(emb_opcode, x_opcode),
        _emb_feat(emb_source, x_source),
        _emb_feat(emb_sink, x_sink),
        _emb_feat(emb_strman, x_string_manip),
        _emb_feat(emb_payload, x_payload),
    ], axis=1)
    h = _layernorm(feats, ln_g, ln_b)
    src, dst = edge_index[0], edge_index[1]
    h = _gat(h, src, dst, W1, att_src1, att_dst1, b1, HEADS, HID)
    h = jax.nn.elu(h)
    h = _gat(h, src, dst, W2, att_src2, att_dst2, b2, 1, HID)
    h = jax.nn.elu(h)
    pooled = jax.ops.segment_max(h, batch, num_segments=G)
    out = pl.pallas_call(
        _mlp_kernel,
        out_shape=jax.ShapeDtypeStruct((G, 2), jnp.float32),
    )(pooled, fc1_w, fc1_b, fc2_w, fc2_b)
    return out


# speedup vs baseline: 17.5225x; 17.5225x over previous
import functools

import jax
import jax.numpy as jnp
from jax import lax
from jax.experimental import pallas as pl
from jax.experimental.pallas import tpu as pltpu
from jax.experimental.pallas import tpu_sc as plsc

N = 50000
E = 800000
EMB = 16
HID = 32
HEADS = 4
G = 64

NC, NS, L = 2, 16, 16
NW = NC * NS
EPW = 26624
EPAD = NW * EPW
NPAD = 50176
CH1 = 2048
NCH1 = EPW // CH1
IG1 = CH1 // 128

_mesh = plsc.VectorSubcoreMesh(core_axis_name="c", subcore_axis_name="s",
                               num_cores=NC, num_subcores=NS)


def _emb_feat(table, idx):
    e = jnp.take(table, idx, axis=0)
    m = (idx != 0)[..., None].astype(jnp.float32)
    return (e * m).sum(axis=1) / (m.sum(axis=1) + 1e-9)


def _layernorm(x, g, b):
    mu = x.mean(-1, keepdims=True)
    var = ((x - mu) ** 2).mean(-1, keepdims=True)
    return (x - mu) / jnp.sqrt(var + 1e-5) * g + b


def _iota16():
    return lax.broadcasted_iota(jnp.int32, (L,), 0)


def _splat(x):
    return jnp.full((L,), x, dtype=jnp.int32)


def _vtake(x, idx):
    return lax.gather(
        x, idx[:, None],
        dimension_numbers=lax.GatherDimensionNumbers(
            offset_dims=(), collapsed_slice_dims=(0,), start_index_map=(0,)),
        slice_sizes=(1,), mode=lax.GatherScatterMode.PROMISE_IN_BOUNDS)


EG1 = CH1 * HEADS // 128


def _sc_p1_body(sv4_hbm, dv4_hbm, asf_hbm, adf_hbm, cvec_hbm, ex_hbm, den_hbm,
                sv4_v, dv4_v, asb, adb, exb, cv, den_sh, sem):
    cid = lax.axis_index("c")
    sid = lax.axis_index("s")
    wid = sid * NC + cid

    zero = jnp.zeros((L,), jnp.float32)

    def zero_body(g, _):
        exb[pl.ds(g * L, L)] = zero
        return _
    lax.fori_loop(0, CH1 * HEADS // L, zero_body, 0)
    stripe = NPAD * HEADS // NS
    nz = CH1 * HEADS
    pltpu.sync_copy(exb, den_sh.at[pl.ds(sid * stripe, nz)])
    pltpu.sync_copy(exb.at[pl.ds(0, stripe - nz)],
                    den_sh.at[pl.ds(sid * stripe + nz, stripe - nz)])
    pltpu.sync_copy(cvec_hbm, cv)
    plsc.subcore_barrier()

    def chunk(t, _):
        ebase = pl.multiple_of((wid * EPW + t * CH1) * HEADS, 1024)
        pltpu.sync_copy(sv4_hbm.at[pl.ds(ebase, CH1 * HEADS)], sv4_v)
        pltpu.sync_copy(dv4_hbm.at[pl.ds(ebase, CH1 * HEADS)], dv4_v)
        ga = pltpu.async_copy(asf_hbm.at[sv4_v], asb, sem)
        gb = pltpu.async_copy(adf_hbm.at[dv4_v], adb, sem)
        ga.wait()
        gb.wait()
        cvv = cv[...]

        def grp(g, _):
            s = asb[pl.ds(g * L, L)] + adb[pl.ds(g * L, L)]
            e_ = jnp.where(s > 0, s, 0.2 * s)
            exb[pl.ds(g * L, L)] = jnp.exp(e_ - cvv)
            return _
        lax.fori_loop(0, CH1 * HEADS // L, grp, 0)
        pltpu.sync_copy(exb, ex_hbm.at[pl.ds(ebase, CH1 * HEADS)])
        pltpu.sync_copy(exb, den_sh.at[dv4_v], add=True)
        return _
    lax.fori_loop(0, NCH1, chunk, 0)

    plsc.subcore_barrier()
    pltpu.sync_copy(den_sh.at[pl.ds(sid * stripe, stripe)],
                    den_hbm.at[pl.ds(cid * (NPAD * HEADS) + sid * stripe,
                                     stripe)])


def _sc_p1(sv4r, dv4r, asf, adf, cvec):
    return pl.kernel(
        _sc_p1_body,
        out_type=(jax.ShapeDtypeStruct((EPAD * HEADS,), jnp.float32),
                  jax.ShapeDtypeStruct((NC * NPAD * HEADS,), jnp.float32)),
        mesh=_mesh,
        compiler_params=pltpu.CompilerParams(use_tc_tiling_on_sc=False,
                                             needs_layout_passes=False),
        scratch_types=[
            pltpu.VMEM((CH1 * HEADS,), jnp.int32),
            pltpu.VMEM((CH1 * HEADS,), jnp.int32),
            pltpu.VMEM((CH1 * HEADS,), jnp.float32),
            pltpu.VMEM((CH1 * HEADS,), jnp.float32),
            pltpu.VMEM((CH1 * HEADS,), jnp.float32),
            pltpu.VMEM((L,), jnp.float32),
            pltpu.VMEM_SHARED((NPAD * HEADS,), jnp.float32),
            pltpu.SemaphoreType.DMA,
        ],
    )(sv4r, dv4r, asf, adf, cvec)


CH2 = 2048
EPS2 = EPAD // NS
NCH2 = EPS2 // CH2


def _sc_p2_body(sv_hbm, dv_hbm, xw_hbm, ex_hbm, num_hbm,
                sv_v, dv_v, xrows, exq, acc_sh, sem):
    cid = lax.axis_index("c")
    sid = lax.axis_index("s")
    i16 = _iota16()
    stripe = NPAD // NS
    zero = jnp.zeros((L,), jnp.float32)

    for p8 in range(8):
        q = p8 // 2

        @pl.when(cid == q // 2)
        def _q():
            def zb(g, _):
                xrows[g, pl.ds(0, 16)] = zero
                return _
            lax.fori_loop(0, CH2, zb, 0)
            pltpu.sync_copy(xrows, acc_sh.at[pl.ds(sid * stripe, CH2)])
            pltpu.sync_copy(xrows.at[pl.ds(0, stripe - CH2)],
                            acc_sh.at[pl.ds(sid * stripe + CH2, stripe - CH2)])
            plsc.subcore_barrier()

            def chunk(t, _):
                base = pl.multiple_of(sid * EPS2 + t * CH2, 1024)
                pltpu.sync_copy(sv_hbm.at[pl.ds(base, CH2)], sv_v)
                pltpu.sync_copy(dv_hbm.at[pl.ds(base, CH2)], dv_v)
                g1 = pltpu.async_copy(xw_hbm.at[p8].at[sv_v], xrows, sem)
                g2 = pltpu.async_copy(
                    ex_hbm.at[pl.ds(base * HEADS, CH2 * HEADS)], exq, sem)
                g1.wait()
                g2.wait()

                def grp(g, _):
                    m16 = plsc.load_gather(
                        exq, [_splat(g * (L * HEADS) + q) + i16 * HEADS])
                    for k in range(L):
                        m = _vtake(m16, _splat(k))
                        e = g * L + k
                        xrows[e, pl.ds(0, 16)] = xrows[e, pl.ds(0, 16)] * m
                    return _
                lax.fori_loop(0, CH2 // L, grp, 0)
                pltpu.sync_copy(xrows, acc_sh.at[dv_v], add=True)
                return _
            lax.fori_loop(0, NCH2, chunk, 0)

            plsc.subcore_barrier()
            pltpu.sync_copy(
                acc_sh.at[pl.ds(sid * stripe, stripe)],
                num_hbm.at[p8].at[pl.ds(sid * stripe, stripe)])
            plsc.subcore_barrier()
    return


def _sc_p2(svp, dvp, xw8, ex1f):
    return pl.kernel(
        _sc_p2_body,
        out_type=jax.ShapeDtypeStruct((8, NPAD, 16), jnp.float32),
        mesh=_mesh,
        compiler_params=pltpu.CompilerParams(use_tc_tiling_on_sc=False,
                                             needs_layout_passes=False),
        scratch_types=[
            pltpu.VMEM((CH2,), jnp.int32),
            pltpu.VMEM((CH2,), jnp.int32),
            pltpu.VMEM((CH2, 16), jnp.float32),
            pltpu.VMEM((CH2 * HEADS,), jnp.float32),
            pltpu.VMEM_SHARED((NPAD, 16), jnp.float32),
            pltpu.SemaphoreType.DMA,
        ],
    )(svp, dvp, xw8, ex1f)


CH3 = 1024
NCH3 = EPS2 // CH3


def _sc_l2_body(sv_hbm, dv_hbm, dvx4_hbm, b2_hbm, adf_hbm, cvec_hbm,
                num_hbm, den_hbm,
                sv_v, dv_v, dvx4_v, brows, outb, adb, denb, cv,
                acc_sh, den_sh, sem):
    cid = lax.axis_index("c")
    sid = lax.axis_index("s")
    i16 = _iota16()
    lane0 = i16 == 0
    stripe = NPAD // NS
    zero = jnp.zeros((L,), jnp.float32)

    def zb(g, _):
        outb[g, pl.ds(0, 16)] = zero
        return _
    lax.fori_loop(0, CH3, zb, 0)

    def zb2(g, _):
        denb[pl.ds(g * L, L)] = zero
        return _
    lax.fori_loop(0, CH3 // L, zb2, 0)
    nfull = stripe // CH3
    for r in range(nfull):
        pltpu.sync_copy(outb.at[pl.ds(0, CH3), :],
                        acc_sh.at[pl.ds(sid * stripe + r * CH3, CH3)])
    rem = stripe - nfull * CH3
    pltpu.sync_copy(outb.at[pl.ds(0, rem), :],
                    acc_sh.at[pl.ds(sid * stripe + nfull * CH3, rem)])

    @pl.when(cid == 0)
    def _zd():
        for r in range(nfull):
            pltpu.sync_copy(denb, den_sh.at[pl.ds(sid * stripe + r * CH3, CH3)])
        pltpu.sync_copy(denb.at[pl.ds(0, rem)],
                        den_sh.at[pl.ds(sid * stripe + nfull * CH3, rem)])
    pltpu.sync_copy(cvec_hbm, cv)
    plsc.subcore_barrier()

    for p in range(2):
        @pl.when(cid == p)
        def _p():
            def chunk(t, _):
                base = pl.multiple_of(sid * EPS2 + t * CH3, 1024)
                pltpu.sync_copy(sv_hbm.at[pl.ds(base, CH3)], sv_v)
                pltpu.sync_copy(dv_hbm.at[pl.ds(base, CH3)], dv_v)
                pltpu.sync_copy(dvx4_hbm.at[pl.ds(base, CH3)], dvx4_v)
                g1 = pltpu.async_copy(b2_hbm.at[p].at[sv_v], brows, sem)
                g2 = pltpu.async_copy(adf_hbm.at[dvx4_v], adb, sem)
                g1.wait()
                g2.wait()
                cvv = cv[...]

                def grp(g, _):
                    ald16 = adb[pl.ds(g * L, L)]
                    for k in range(L):
                        e = g * L + k
                        v = brows[e, pl.ds(16, 16)]
                        s = _vtake(v, _splat(0)) + _vtake(ald16, _splat(k))
                        e_ = jnp.where(s > 0, s, 0.2 * s)
                        ex = jnp.exp(e_ - cvv)
                        outb[e, pl.ds(0, 16)] = brows[e, pl.ds(0, 16)] * ex
                        if p == 0:
                            plsc.store_scatter(denb, [_splat(e)], ex,
                                               mask=lane0)
                    return _
                lax.fori_loop(0, CH3 // L, grp, 0)
                pltpu.sync_copy(outb, acc_sh.at[dv_v], add=True)
                if p == 0:
                    pltpu.sync_copy(denb, den_sh.at[dv_v], add=True)
                return _
            lax.fori_loop(0, NCH3, chunk, 0)

            plsc.subcore_barrier()
            pltpu.sync_copy(acc_sh.at[pl.ds(sid * stripe, stripe)],
                            num_hbm.at[p].at[pl.ds(sid * stripe, stripe)])
            if p == 0:
                pltpu.sync_copy(den_sh.at[pl.ds(sid * stripe, stripe)],
                                den_hbm.at[pl.ds(sid * stripe, stripe)])


def _sc_l2(svp, dvp, dvx4, b2, adf2, cvec):
    return pl.kernel(
        _sc_l2_body,
        out_type=(jax.ShapeDtypeStruct((2, NPAD, 16), jnp.float32),
                  jax.ShapeDtypeStruct((NPAD,), jnp.float32)),
        mesh=_mesh,
        compiler_params=pltpu.CompilerParams(use_tc_tiling_on_sc=False,
                                             needs_layout_passes=False),
        scratch_types=[
            pltpu.VMEM((CH3,), jnp.int32),
            pltpu.VMEM((CH3,), jnp.int32),
            pltpu.VMEM((CH3,), jnp.int32),
            pltpu.VMEM((CH3, 32), jnp.float32),
            pltpu.VMEM((CH3, 16), jnp.float32),
            pltpu.VMEM((CH3,), jnp.float32),
            pltpu.VMEM((CH3,), jnp.float32),
            pltpu.VMEM((L,), jnp.float32),
            pltpu.VMEM_SHARED((NPAD, 16), jnp.float32),
            pltpu.VMEM_SHARED((NPAD,), jnp.float32),
            pltpu.SemaphoreType.DMA,
        ],
    )(svp, dvp, dvx4, b2, adf2, cvec)


BK = 256
NBK = NPAD // BK


def _onehot_pool(idx, table, rows):
    v = table.shape[0]
    vio = lax.broadcasted_iota(jnp.int32, (1, v), 1)
    cnt = jnp.zeros((rows, v), jnp.float32)
    for slot in range(16):
        col = idx[:, slot:slot + 1]
        cnt = cnt + jnp.where((col == vio) & (col != 0), 1.0, 0.0)
    nnz = jnp.sum(jnp.where(idx != 0, 1.0, 0.0), axis=1,
                  keepdims=True)
    return (cnt @ table) / (nnz + 1e-9)


def _k1_body(xs, xo, xsrc, xsink, xstr, xpay,
             t_op, t_src, t_sink, t_str, t_pay,
             lng, lnb, w1, as1, ad1,
             xw4_o, asf_o, adf_o, mx_o):
    i = pl.program_id(0)
    feats = jnp.concatenate([
        xs[...],
        _onehot_pool(xo[...], t_op[...], BK),
        _onehot_pool(xsrc[...], t_src[...], BK),
        _onehot_pool(xsink[...], t_sink[...], BK),
        _onehot_pool(xstr[...], t_str[...], BK),
        _onehot_pool(xpay[...], t_pay[...], BK),
    ], axis=1)
    mu = feats.mean(-1, keepdims=True)
    var = ((feats - mu) ** 2).mean(-1, keepdims=True)
    h = (feats - mu) / jnp.sqrt(var + 1e-5) * lng[...] + lnb[...]
    xw = h @ w1[...]
    xwr = xw.reshape(BK, HEADS, HID)
    als = (xwr * as1[...][None]).sum(-1)
    ald = (xwr * ad1[...][None]).sum(-1)
    for p in range(8):
        xw4_o[p, :, :] = xw[:, p * 16:(p + 1) * 16]
    asf_o[...] = als
    adf_o[...] = ald
    rid = i * BK + lax.broadcasted_iota(jnp.int32, (BK, 1), 0)
    m = rid < N
    mx_o[...] = jnp.concatenate([
        jnp.max(jnp.where(m, als, -jnp.inf), axis=0),
        jnp.max(jnp.where(m, ald, -jnp.inf), axis=0)])[None, None]


def _k1(x_scalar, x_opcode, x_source, x_sink, x_str, x_pay,
        t_op, t_src, t_sink, t_str, t_pay, lng, lnb, w1, as1, ad1):
    blk = lambda r: pl.BlockSpec((BK, r), lambda i: (i, 0))
    full = lambda a: pl.BlockSpec(a.shape, lambda i: (0,) * a.ndim)
    return pl.pallas_call(
        _k1_body,
        grid=(NBK,),
        in_specs=[blk(16)] * 6 + [full(t) for t in
                                  (t_op, t_src, t_sink, t_str, t_pay)]
        + [full(lng), full(lnb), full(w1), full(as1), full(ad1)],
        out_specs=(pl.BlockSpec((8, BK, 16), lambda i: (0, i, 0)),
                   pl.BlockSpec((BK, HEADS), lambda i: (i, 0)),
                   pl.BlockSpec((BK, HEADS), lambda i: (i, 0)),
                   pl.BlockSpec((1, 1, 2 * HEADS), lambda i: (i, 0, 0))),
        out_shape=(jax.ShapeDtypeStruct((8, NPAD, 16), jnp.float32),
                   jax.ShapeDtypeStruct((NPAD, HEADS), jnp.float32),
                   jax.ShapeDtypeStruct((NPAD, HEADS), jnp.float32),
                   jax.ShapeDtypeStruct((NBK, 1, 2 * HEADS), jnp.float32)),
    )(x_scalar, x_opcode, x_source, x_sink, x_str, x_pay,
      t_op, t_src, t_sink, t_str, t_pay, lng, lnb, w1, as1, ad1)


def _k4_body(num1, den1, asf, adf, xw4, c1, b1, w2, as2, ad2,
             b2p_o, adf2_o, mx_o):
    i = pl.program_id(0)
    als = asf[...]
    ald = adf[...]
    s = als + ald
    selfex = jnp.exp(jnp.where(s > 0, s, 0.2 * s) - c1[...])
    den = den1[0] + den1[1] + selfex
    parts = []
    for q in range(HEADS):
        numq = (jnp.concatenate([num1[2 * q], num1[2 * q + 1]], axis=1)
                + jnp.concatenate([xw4[2 * q], xw4[2 * q + 1]], axis=1)
                * selfex[:, q:q + 1])
        outq = numq / (den[:, q:q + 1] + 1e-16) + b1[...][None, q * HID:(q + 1) * HID]
        parts.append(outq)
    out1 = jnp.concatenate(parts, axis=1)
    h2 = jnp.where(out1 > 0, out1, jnp.exp(out1) - 1.0)
    xw2 = h2 @ w2[...]
    als2 = (xw2 * as2[...]).sum(-1)
    ald2 = (xw2 * ad2[...]).sum(-1)
    for p in range(2):
        b2p_o[p, :, :] = jnp.concatenate(
            [xw2[:, p * 16:(p + 1) * 16], als2[:, None],
             jnp.zeros((BK, 15), jnp.float32)], axis=1)
    adf2_o[...] = jnp.concatenate(
        [ald2[:, None], jnp.zeros((BK, 3), jnp.float32)], axis=1)
    rid = i * BK + lax.broadcasted_iota(jnp.int32, (BK,), 0)
    m = rid < N
    mx_o[...] = jnp.concatenate([
        jnp.max(jnp.where(m, als2, -jnp.inf))[None],
        jnp.max(jnp.where(m, ald2, -jnp.inf))[None],
        jnp.zeros((6,), jnp.float32)])[None, None]


def _k4(num1p, den1p, asf, adf, xw4, c1, b1, w2, as2, ad2):
    full = lambda a: pl.BlockSpec(a.shape, lambda i: (0,) * a.ndim)
    return pl.pallas_call(
        _k4_body,
        grid=(NBK,),
        in_specs=[pl.BlockSpec((8, BK, 16), lambda i: (0, i, 0)),
                  pl.BlockSpec((NC, BK, HEADS), lambda i: (0, i, 0)),
                  pl.BlockSpec((BK, HEADS), lambda i: (i, 0)),
                  pl.BlockSpec((BK, HEADS), lambda i: (i, 0)),
                  pl.BlockSpec((8, BK, 16), lambda i: (0, i, 0)),
                  pl.BlockSpec((1, HEADS), lambda i: (0, 0)),
                  full(b1), full(w2), full(as2), full(ad2)],
        out_specs=(pl.BlockSpec((2, BK, 32), lambda i: (0, i, 0)),
                   pl.BlockSpec((BK, 4), lambda i: (i, 0)),
                   pl.BlockSpec((1, 1, 8), lambda i: (i, 0, 0))),
        out_shape=(jax.ShapeDtypeStruct((2, NPAD, 32), jnp.float32),
                   jax.ShapeDtypeStruct((NPAD, 4), jnp.float32),
                   jax.ShapeDtypeStruct((NBK, 1, 8), jnp.float32)),
    )(num1p, den1p, asf, adf, xw4, c1, b1, w2, as2, ad2)


def _k5_body(num2, den2, b2p, adf2, c2, b2, batch,
             fw1, fb1, fw2, fb2, out_o, pooled):
    i = pl.program_id(0)

    @pl.when(i == 0)
    def _init():
        pooled[...] = jnp.full((G, HID), -jnp.inf, jnp.float32)

    xw2 = jnp.concatenate([b2p[0, :, :16], b2p[1, :, :16]], axis=1)
    als2 = b2p[0, :, 16]
    ald2 = adf2[:, 0]
    s = als2 + ald2
    selfex = jnp.exp(jnp.where(s > 0, s, 0.2 * s) - c2[0, 0])
    den = den2[:, 0] + selfex
    num = (jnp.concatenate([num2[0], num2[1]], axis=1)
           + xw2 * selfex[:, None])
    out2 = num / (den[:, None] + 1e-16) + b2[...][None]
    h3 = jnp.where(out2 > 0, out2, jnp.exp(out2) - 1.0)
    rid = i * BK + lax.broadcasted_iota(jnp.int32, (BK, 1), 0)
    m = rid < N
    bcol = batch[...]
    for g in range(G):
        mg = (bcol == g) & m
        cg = jnp.max(jnp.where(mg, h3, -jnp.inf), axis=0,
                     keepdims=True)
        pooled[pl.ds(g, 1), :] = jnp.maximum(pooled[pl.ds(g, 1), :], cg)

    @pl.when(i == NBK - 1)
    def _fin():
        hm = jnp.maximum(pooled[...] @ fw1[...] + fb1[...][None], 0.0)
        out_o[...] = hm @ fw2[...] + fb2[...][None]


def _k5(num2p, den2p, b2p, adf2, c2, b2, batchp, fw1, fb1, fw2, fb2):
    full = lambda a: pl.BlockSpec(a.shape, lambda i: (0,) * a.ndim)
    return pl.pallas_call(
        _k5_body,
        grid=(NBK,),
        in_specs=[pl.BlockSpec((2, BK, 16), lambda i: (0, i, 0)),
                  pl.BlockSpec((BK, 1), lambda i: (i, 0)),
                  pl.BlockSpec((2, BK, 32), lambda i: (0, i, 0)),
                  pl.BlockSpec((BK, 4), lambda i: (i, 0)),
                  pl.BlockSpec((1, 1), lambda i: (0, 0)),
                  full(b2),
                  pl.BlockSpec((BK, 1), lambda i: (i, 0)),
                  full(fw1), full(fb1), full(fw2), full(fb2)],
        out_specs=pl.BlockSpec((G, 2), lambda i: (0, 0)),
        out_shape=jax.ShapeDtypeStruct((G, 2), jnp.float32),
        scratch_shapes=[pltpu.VMEM((G, HID), jnp.float32)],
    )(num2p, den2p, b2p, adf2, c2, b2, batchp, fw1, fb1, fw2, fb2)


def _mlp_kernel(pooled_ref, w1_ref, b1_ref, w2_ref, b2_ref, out_ref):
    h = jnp.maximum(pooled_ref[...] @ w1_ref[...] + b1_ref[...], 0.0)
    out_ref[...] = h @ w2_ref[...] + b2_ref[...]


def kernel(x_scalar, x_opcode, x_source, x_sink, x_string_manip, x_payload, edge_index, batch, emb_opcode, emb_source, emb_sink, emb_strman, emb_payload, ln_g, ln_b, W1, att_src1, att_dst1, b1, W2, att_src2, att_dst2, b2, fc1_w, fc1_b, fc2_w, fc2_b):
    src, dst = edge_index[0], edge_index[1]

    padi = jnp.arange(EPAD - E, dtype=jnp.int32) % 16
    svp = jnp.concatenate([src.astype(jnp.int32), padi])
    dvp = jnp.concatenate([dst.astype(jnp.int32), N + padi])
    h4 = jnp.arange(4, dtype=jnp.int32)[None]
    sv4r = (svp[:, None] * 4 + h4).reshape(-1)
    dv4r = (dvp[:, None] * 4 + h4).reshape(-1)
    dvx4 = dvp * 4

    xw8, asf, adf, mx1 = _k1(
        x_scalar, x_opcode.astype(jnp.int32), x_source.astype(jnp.int32),
        x_sink.astype(jnp.int32), x_string_manip.astype(jnp.int32),
        x_payload.astype(jnp.int32),
        emb_opcode, emb_source, emb_sink, emb_strman, emb_payload,
        ln_g, ln_b, W1, att_src1, att_dst1)
    mxs = mx1.reshape(NBK, 8).max(0)
    c1r = mxs[:4] + mxs[4:]
    c1 = jnp.where(c1r > 0, c1r, 0.2 * c1r)
    cvec1 = jnp.concatenate([c1, c1, c1, c1])

    ex1f, den1f = _sc_p1(sv4r, dv4r, asf.reshape(-1), adf.reshape(-1), cvec1)
    num1p = _sc_p2(svp, dvp, xw8, ex1f)
    den1p = den1f.reshape(NC, NPAD, HEADS)

    b2p, adf2, mx2 = _k4(num1p, den1p, asf, adf, xw8, c1[None],
                         b1, W2, att_src2, att_dst2)
    m2 = mx2.reshape(NBK, 8).max(0)
    c2r = m2[0] + m2[1]
    c2 = jnp.where(c2r > 0, c2r, 0.2 * c2r)
    cvec2 = jnp.full((16,), c2, jnp.float32)

    num2p, den2f = _sc_l2(svp, dvp, dvx4, b2p, adf2.reshape(-1), cvec2)
    den2p = den2f.reshape(NPAD, 1)

    batchp = jnp.concatenate(
        [batch.astype(jnp.int32), jnp.zeros((NPAD - N,), jnp.int32)]
    ).reshape(NPAD, 1)
    out = _k5(num2p, den2p, b2p, adf2, c2.reshape(1, 1), b2, batchp,
              fc1_w, fc1_b, fc2_w, fc2_b)
    return out
